# baseline (speedup 1.0000x reference)
import functools

import jax
import jax.numpy as jnp
from jax import lax
from jax.experimental import pallas as pl
from jax.experimental.pallas import tpu as pltpu
from jax.experimental.pallas import tpu_sc as plsc

_D = 256
_N = 18432
_NSC_ROWS = _N // 2
_SC_ROW0 = _N - _NSC_ROWS
_NC, _NS = 2, 16
_NW = _NC * _NS
_ROWS_W = _NSC_ROWS // _NW
_L = 16
_NG = _D // _L

_TC_BR = 1152
_TC_STEPS = _N // _TC_BR
_TC_RED = (_N - _NSC_ROWS) // _TC_BR


def _sc_body(x_hbm, w_hbm, part_hbm, idx_hbm, w0_v, x_v, idx_v, acc_v):
    wid = lax.axis_index("s") * _NC + lax.axis_index("c")
    row0 = _SC_ROW0 + wid * _ROWS_W

    pltpu.sync_copy(w_hbm.at[0], w0_v)
    w0s = [w0_v[pl.ds(_L * j, _L)] for j in range(_NG)]

    zi = jnp.zeros((_L,), jnp.int32)

    def _zfill(r, carry):
        idx_v[pl.ds(r * _L, _L)] = zi
        return carry
    lax.fori_loop(0, (_N // _NW) // _L, _zfill, 0)
    pltpu.sync_copy(idx_v, idx_hbm.at[pl.ds(wid * (_N // _NW), _N // _NW)])

    pltpu.sync_copy(x_hbm.at[pl.ds(row0, _ROWS_W), :], x_v)
    accs = tuple(jnp.zeros((_L,), jnp.float32) for _ in range(_NG))

    def _row(r, accs):
        out = []
        for j in range(_NG):
            d = x_v[r, pl.ds(_L * j, _L)] - w0s[j]
            out.append(accs[j] + d * d)
        return tuple(out)
    accs = lax.fori_loop(0, _ROWS_W, _row, accs)

    acc = accs[0]
    for j in range(1, _NG):
        acc = acc + accs[j]
    acc_v[...] = acc
    pltpu.sync_copy(acc_v, part_hbm.at[wid])


_sc_kernel = functools.partial(
    pl.kernel,
    out_type=[
        jax.ShapeDtypeStruct((_NW, _L), jnp.float32),
        jax.ShapeDtypeStruct((_N,), jnp.int32),
    ],
    mesh=plsc.VectorSubcoreMesh(core_axis_name="c", subcore_axis_name="s",
                                num_cores=_NC, num_subcores=_NS),
    scratch_types=[
        pltpu.VMEM((_D,), jnp.float32),
        pltpu.VMEM((_ROWS_W, _D), jnp.float32),
        pltpu.VMEM((_N // _NW,), jnp.int32),
        pltpu.VMEM((_L,), jnp.float32),
    ],
)(_sc_body)


def _tc_body(x_ref, w_ref, q_ref, sse_ref, acc_ref):
    i = pl.program_id(0)
    w0 = w_ref[0:1, :]

    @pl.when(i < _TC_RED)
    def _reduce_and_write():
        x = x_ref[...]
        d = w0 - x
        q_ref[...] = jnp.broadcast_to(w0, (_TC_BR, _D))
        part = jnp.sum(d * d)

        @pl.when(i == 0)
        def _init():
            acc_ref[0, 0] = part

        @pl.when(i > 0)
        def _acc():
            acc_ref[0, 0] += part

    @pl.when(i >= _TC_RED)
    def _write_only():
        q_ref[...] = jnp.broadcast_to(w0, (_TC_BR, _D))

    @pl.when(i == _TC_STEPS - 1)
    def _fin():
        sse_ref[...] = jnp.full((1, 1), acc_ref[0, 0], jnp.float32)


def _combine_body(p_ref, sse_ref, loss_ref, perp_ref):
    sse = jnp.sum(p_ref[...]) + sse_ref[0, 0]
    loss = sse * (jnp.float32(1.25) / jnp.float32(_N * _D))
    loss_ref[...] = jnp.full((1, 1), loss, jnp.float32)
    perp = jnp.exp(-(jnp.log(jnp.float32(1.0) + jnp.float32(1e-10))))
    perp_ref[...] = jnp.full((1, 1), perp, jnp.float32)


def kernel(inputs, W):
    shape = inputs.shape
    flat = inputs.reshape(-1, _D)

    q, sse = pl.pallas_call(
        _tc_body,
        grid=(_TC_STEPS,),
        in_specs=[
            pl.BlockSpec((_TC_BR, _D), lambda i: (jnp.minimum(i, _TC_RED - 1), 0)),
            pl.BlockSpec((8, _D), lambda i: (0, 0)),
        ],
        out_specs=[
            pl.BlockSpec((_TC_BR, _D), lambda i: (i, 0)),
            pl.BlockSpec((1, 1), lambda i: (0, 0)),
        ],
        out_shape=[
            jax.ShapeDtypeStruct((_N, _D), jnp.float32),
            jax.ShapeDtypeStruct((1, 1), jnp.float32),
        ],
        scratch_shapes=[pltpu.SMEM((1, 1), jnp.float32)],
    )(flat, W)

    part, idx = _sc_kernel(flat, W)

    loss, perp = pl.pallas_call(
        _combine_body,
        out_shape=[
            jax.ShapeDtypeStruct((1, 1), jnp.float32),
            jax.ShapeDtypeStruct((1, 1), jnp.float32),
        ],
    )(part, sse)

    return (q.reshape(shape), loss.reshape(()), perp.reshape(()),
            idx.reshape(shape[:2]))


# speedup vs baseline: 1.0603x; 1.0603x over previous
import functools

import jax
import jax.numpy as jnp
from jax import lax
from jax.experimental import pallas as pl
from jax.experimental.pallas import tpu as pltpu
from jax.experimental.pallas import tpu_sc as plsc

_D = 256
_N = 18432
_NC, _NS = 2, 16
_NW = _NC * _NS
_L = 16
_BR = 2048
_STEPS = _N // _BR


def _sc_body(idx_hbm, idx_v):
    wid = lax.axis_index("s") * _NC + lax.axis_index("c")
    nw = _N // _NW
    zi = jnp.zeros((_L,), jnp.int32)

    def _zfill(r, carry):
        idx_v[pl.ds(r * _L, _L)] = zi
        return carry
    lax.fori_loop(0, nw // _L, _zfill, 0)
    pltpu.sync_copy(idx_v, idx_hbm.at[pl.ds(wid * nw, nw)])


_sc_kernel = functools.partial(
    pl.kernel,
    out_type=[jax.ShapeDtypeStruct((_N,), jnp.int32)],
    mesh=plsc.VectorSubcoreMesh(core_axis_name="c", subcore_axis_name="s",
                                num_cores=_NC, num_subcores=_NS),
    scratch_types=[pltpu.VMEM((_N // _NW,), jnp.int32)],
)(_sc_body)


def _vq_body(x_ref, w_ref, q_ref, loss_ref, perp_ref, acc_ref):
    i = pl.program_id(0)
    w0 = w_ref[0:1, :]
    x = x_ref[...]
    d = w0 - x
    q_ref[...] = jnp.broadcast_to(w0, (_BR, _D))
    part = jnp.sum(d * d)

    @pl.when(i == 0)
    def _init():
        acc_ref[0, 0] = part
        perp = jnp.exp(-(jnp.log(jnp.float32(1.0) + jnp.float32(1e-10))))
        perp_ref[...] = jnp.full((1, 1), perp, jnp.float32)

    @pl.when(i > 0)
    def _acc():
        acc_ref[0, 0] += part

    @pl.when(i == _STEPS - 1)
    def _fin():
        loss = acc_ref[0, 0] * (jnp.float32(1.25) / jnp.float32(_N * _D))
        loss_ref[...] = jnp.full((1, 1), loss, jnp.float32)


def kernel(inputs, W):
    shape = inputs.shape
    flat = inputs.reshape(-1, _D)

    (idx,) = _sc_kernel()

    q, loss, perp = pl.pallas_call(
        _vq_body,
        grid=(_STEPS,),
        in_specs=[
            pl.BlockSpec((_BR, _D), lambda i: (i, 0)),
            pl.BlockSpec((8, _D), lambda i: (0, 0)),
        ],
        out_specs=[
            pl.BlockSpec((_BR, _D), lambda i: (i, 0)),
            pl.BlockSpec((1, 1), lambda i: (0, 0)),
            pl.BlockSpec((1, 1), lambda i: (0, 0)),
        ],
        out_shape=[
            jax.ShapeDtypeStruct((_N, _D), jnp.float32),
            jax.ShapeDtypeStruct((1, 1), jnp.float32),
            jax.ShapeDtypeStruct((1, 1), jnp.float32),
        ],
        scratch_shapes=[pltpu.SMEM((1, 1), jnp.float32)],
    )(flat, W)

    return (q.reshape(shape), loss.reshape(()), perp.reshape(()),
            idx.reshape(shape[:2]))


# speedup vs baseline: 2.0689x; 1.9512x over previous
import jax
import jax.numpy as jnp
from jax.experimental import pallas as pl
from jax.experimental.pallas import tpu as pltpu

_D = 256
_N = 18432
_BR = 3072
_STEPS = _N // _BR
_IC = 128
_IBR = _BR // _IC


def _vq_body(x_ref, w_ref, q_ref, idx_ref, loss_ref, perp_ref, acc_ref):
    i = pl.program_id(0)
    w0 = w_ref[0:1, :]
    x = x_ref[...]
    d = w0 - x
    q_ref[...] = jnp.broadcast_to(w0, (_BR, _D))
    idx_ref[...] = jnp.zeros((_IBR, _IC), jnp.int32)
    part = jnp.sum(d * d)

    @pl.when(i == 0)
    def _init():
        acc_ref[0, 0] = part
        perp = jnp.exp(-(jnp.log(jnp.float32(1.0) + jnp.float32(1e-10))))
        perp_ref[...] = jnp.full((1, 1), perp, jnp.float32)

    @pl.when(i > 0)
    def _acc():
        acc_ref[0, 0] += part

    @pl.when(i == _STEPS - 1)
    def _fin():
        loss = acc_ref[0, 0] * (jnp.float32(1.25) / jnp.float32(_N * _D))
        loss_ref[...] = jnp.full((1, 1), loss, jnp.float32)


def kernel(inputs, W):
    shape = inputs.shape
    flat = inputs.reshape(-1, _D)

    q, idx, loss, perp = pl.pallas_call(
        _vq_body,
        grid=(_STEPS,),
        in_specs=[
            pl.BlockSpec((_BR, _D), lambda i: (i, 0)),
            pl.BlockSpec((8, _D), lambda i: (0, 0)),
        ],
        out_specs=[
            pl.BlockSpec((_BR, _D), lambda i: (i, 0)),
            pl.BlockSpec((_IBR, _IC), lambda i: (i, 0)),
            pl.BlockSpec((1, 1), lambda i: (0, 0)),
            pl.BlockSpec((1, 1), lambda i: (0, 0)),
        ],
        out_shape=[
            jax.ShapeDtypeStruct((_N, _D), jnp.float32),
            jax.ShapeDtypeStruct((_N // _IC, _IC), jnp.int32),
            jax.ShapeDtypeStruct((1, 1), jnp.float32),
            jax.ShapeDtypeStruct((1, 1), jnp.float32),
        ],
        scratch_shapes=[pltpu.SMEM((1, 1), jnp.float32)],
    )(flat, W)

    return (q.reshape(shape), loss.reshape(()), perp.reshape(()),
            idx.reshape(shape[:2]))


# speedup vs baseline: 2.0770x; 1.0039x over previous
import jax
import jax.numpy as jnp
from jax.experimental import pallas as pl
from jax.experimental.pallas import tpu as pltpu

_D = 256
_N = 18432
_BR = 6144
_STEPS = _N // _BR
_IC = 128
_IBR = _BR // _IC


def _vq_body(x_ref, w_ref, q_ref, idx_ref, loss_ref, perp_ref, acc_ref):
    i = pl.program_id(0)
    w0 = w_ref[0:1, :]
    x = x_ref[...]
    d = w0 - x
    q_ref[...] = jnp.broadcast_to(w0, (_BR, _D))
    idx_ref[...] = jnp.zeros((_IBR, _IC), jnp.int32)
    part = jnp.sum(d * d)

    @pl.when(i == 0)
    def _init():
        acc_ref[0, 0] = part
        perp = jnp.exp(-(jnp.log(jnp.float32(1.0) + jnp.float32(1e-10))))
        perp_ref[...] = jnp.full((1, 1), perp, jnp.float32)

    @pl.when(i > 0)
    def _acc():
        acc_ref[0, 0] += part

    @pl.when(i == _STEPS - 1)
    def _fin():
        loss = acc_ref[0, 0] * (jnp.float32(1.25) / jnp.float32(_N * _D))
        loss_ref[...] = jnp.full((1, 1), loss, jnp.float32)


def kernel(inputs, W):
    shape = inputs.shape
    flat = inputs.reshape(-1, _D)

    q, idx, loss, perp = pl.pallas_call(
        _vq_body,
        grid=(_STEPS,),
        in_specs=[
            pl.BlockSpec((_BR, _D), lambda i: (i, 0)),
            pl.BlockSpec((8, _D), lambda i: (0, 0)),
        ],
        out_specs=[
            pl.BlockSpec((_BR, _D), lambda i: (i, 0)),
            pl.BlockSpec((_IBR, _IC), lambda i: (i, 0)),
            pl.BlockSpec((1, 1), lambda i: (0, 0)),
            pl.BlockSpec((1, 1), lambda i: (0, 0)),
        ],
        out_shape=[
            jax.ShapeDtypeStruct((_N, _D), jnp.float32),
            jax.ShapeDtypeStruct((_N // _IC, _IC), jnp.int32),
            jax.ShapeDtypeStruct((1, 1), jnp.float32),
            jax.ShapeDtypeStruct((1, 1), jnp.float32),
        ],
        scratch_shapes=[pltpu.SMEM((1, 1), jnp.float32)],
    )(flat, W)

    return (q.reshape(shape), loss.reshape(()), perp.reshape(()),
            idx.reshape(shape[:2]))


# speedup vs baseline: 2.1231x; 1.0222x over previous
import jax
import jax.numpy as jnp
from jax.experimental import pallas as pl
from jax.experimental.pallas import tpu as pltpu

_D = 256
_N = 18432
_BR = 3072
_STEPS = _N // _BR
_IC = 128
_IBR = _BR // _IC


def _vq_body(x_ref, w_ref, q_ref, idx_ref, sse_ref):
    w0 = w_ref[0:1, :]
    x = x_ref[...]
    d = w0 - x
    q_ref[...] = jnp.broadcast_to(w0, (_BR, _D))
    idx_ref[...] = jnp.zeros((_IBR, _IC), jnp.int32)
    sse_ref[...] = jnp.full((8, 128), jnp.sum(d * d), jnp.float32)


def _combine_body(p_ref, loss_ref, perp_ref):
    loss = jnp.sum(p_ref[...]) * (
        jnp.float32(1.25) / jnp.float32(_N * _D) / jnp.float32(8 * 128))
    loss_ref[...] = jnp.full((1, 1), loss, jnp.float32)
    perp = jnp.exp(-(jnp.log(jnp.float32(1.0) + jnp.float32(1e-10))))
    perp_ref[...] = jnp.full((1, 1), perp, jnp.float32)


def kernel(inputs, W):
    shape = inputs.shape
    flat = inputs.reshape(-1, _D)

    q, idx, parts = pl.pallas_call(
        _vq_body,
        grid=(_STEPS,),
        in_specs=[
            pl.BlockSpec((_BR, _D), lambda i: (i, 0)),
            pl.BlockSpec((8, _D), lambda i: (0, 0)),
        ],
        out_specs=[
            pl.BlockSpec((_BR, _D), lambda i: (i, 0)),
            pl.BlockSpec((_IBR, _IC), lambda i: (i, 0)),
            pl.BlockSpec((8, 128), lambda i: (i, 0)),
        ],
        out_shape=[
            jax.ShapeDtypeStruct((_N, _D), jnp.float32),
            jax.ShapeDtypeStruct((_N // _IC, _IC), jnp.int32),
            jax.ShapeDtypeStruct((_STEPS * 8, 128), jnp.float32),
        ],
        compiler_params=pltpu.CompilerParams(
            dimension_semantics=("parallel",)),
    )(flat, W)

    loss, perp = pl.pallas_call(
        _combine_body,
        out_shape=[
            jax.ShapeDtypeStruct((1, 1), jnp.float32),
            jax.ShapeDtypeStruct((1, 1), jnp.float32),
        ],
    )(parts)

    return (q.reshape(shape), loss.reshape(()), perp.reshape(()),
            idx.reshape(shape[:2]))
